# Initial kernel scaffold; baseline (speedup 1.0000x reference)
import jax
import jax.numpy as jnp
from jax.experimental import pallas as pl


def kernel(x, edge_index, edge_attr, Wh, bh, Wf, bf, Wg, bg, bn_w, bn_b):
    raise NotImplementedError("write your pallas kernel here")


# speedup vs baseline: 3.7752x; 3.7752x over previous
import functools

import jax
import jax.numpy as jnp
from jax import lax
from jax.experimental import pallas as pl
from jax.experimental.pallas import tpu as pltpu
from jax.experimental.pallas import tpu_sc as plsc

_N = 50000
_OUT = 128
_R = 784
_NPAD = 64 * _R
_EBLK = 2048
_G = 128
_PBUF = 2576
_MLOC = (_R + 1) * _OUT
_BLK = 512
_NB = _NPAD // _BLK

_mesh = plsc.VectorSubcoreMesh(core_axis_name="c", subcore_axis_name="s")


def _seg_max_sc(nblk):
    @functools.partial(
        pl.kernel,
        mesh=_mesh,
        compiler_params=pltpu.CompilerParams(
            needs_layout_passes=False, use_tc_tiling_on_sc=False),
        out_type=jax.ShapeDtypeStruct((_NPAD * _OUT,), jnp.float32),
        scratch_types=[
            pltpu.VMEM((_MLOC,), jnp.float32),
            pltpu.VMEM((_EBLK,), jnp.int32),
            pltpu.VMEM((_EBLK,), jnp.int32),
            pltpu.VMEM((_EBLK,), jnp.int32),
            pltpu.VMEM((_EBLK,), jnp.int32),
            pltpu.VMEM((_PBUF,), jnp.int32),
            pltpu.VMEM((_PBUF,), jnp.int32),
            pltpu.VMEM((_G, 16), jnp.float32),
            pltpu.VMEM((512,), jnp.float32),
            pltpu.SemaphoreType.DMA,
            pltpu.SemaphoreType.DMA,
            pltpu.SemaphoreType.DMA,
            pltpu.SemaphoreType.DMA,
            pltpu.SemaphoreType.DMA,
        ],
    )
    def sc_fn(src_hbm, dst_hbm, ftab_hbm, wft_hbm, out_hbm,
              m_loc, dbuf0, dbuf1, sbuf0, sbuf1, mt_src, mt_dst, rows, wf_v,
              sd0, sd1, ss0, ss1, gsem):
        wid = lax.axis_index("s") * 2 + lax.axis_index("c")
        pltpu.sync_copy(wft_hbm, wf_v)
        ws = [wf_v[pl.ds(t * 16, 16)] for t in range(32)]
        iota = lax.iota(jnp.int32, 16)
        neg = jnp.full((16,), -jnp.inf, jnp.float32)
        zero16 = jnp.zeros((16,), jnp.int32)
        trash16 = jnp.full((16,), _R, jnp.int32)
        sd = [sd0, sd1]
        ss = [ss0, ss1]
        dbuf = [dbuf0, dbuf1]
        sbuf = [sbuf0, sbuf1]

        def do_chunk(off):
            pltpu.async_copy(ftab_hbm.at[mt_src.at[pl.ds(off, _G)]],
                             rows, gsem).wait()

            def group_body(g, _):
                dl16 = mt_dst[pl.ds(off + g * 16, 16)]
                for lane in range(16):
                    rv = rows[g * 16 + lane, :]
                    fx = rv[0]
                    fy = rv[1]
                    fz = rv[2]
                    fw = rv[3]
                    base = dl16[lane] * _OUT
                    for c8 in range(8):
                        acc = (ws[c8] * fx + ws[8 + c8] * fy
                               + ws[16 + c8] * fz + ws[24 + c8] * fw)
                        sl = pl.ds(base + c8 * 16, 16)
                        m_loc[sl] = jnp.maximum(m_loc[sl], acc)
                return 0

            lax.fori_loop(0, _G // 16, group_body, 0)

        for r2 in range(2):
            rng_list = [lambda w: w * 2, lambda w: w * 2 + 1]
            rng = rng_list[r2](wid)
            lo = rng * _R
            hi = lo + _R

            def init_body(i, _):
                m_loc[pl.ds(i * 16, 16)] = neg
                return 0

            lax.fori_loop(0, _MLOC // 16, init_body, 0)

            pltpu.async_copy(dst_hbm.at[pl.ds(0, _EBLK)], dbuf[0], sd[0])
            pltpu.async_copy(src_hbm.at[pl.ds(0, _EBLK)], sbuf[0], ss[0])

            def do_block(b, slot, cnt, lo=lo, hi=hi):
                pltpu.make_async_copy(dst_hbm.at[pl.ds(0, _EBLK)],
                                      dbuf[slot], sd[slot]).wait()
                pltpu.make_async_copy(src_hbm.at[pl.ds(0, _EBLK)],
                                      sbuf[slot], ss[slot]).wait()

                @pl.when(b + 1 < nblk)
                def _():
                    nb = (b + 1) * _EBLK
                    pltpu.async_copy(dst_hbm.at[pl.ds(nb, _EBLK)],
                                     dbuf[1 - slot], sd[1 - slot])
                    pltpu.async_copy(src_hbm.at[pl.ds(nb, _EBLK)],
                                     sbuf[1 - slot], ss[1 - slot])

                dref = dbuf[slot]
                sref = sbuf[slot]

                def scan_body(i, cnt):
                    d = dref[pl.ds(i * 16, 16)]
                    msk = (d >= lo) & (d < hi)

                    def matched(cnt):
                        s = sref[pl.ds(i * 16, 16)]
                        mi = msk.astype(jnp.int32)
                        csum = plsc.cumsum(mi)
                        pos = cnt + csum - mi
                        plsc.store_scatter(mt_dst, [pos], d - lo, mask=msk)
                        plsc.store_scatter(mt_src, [pos], s, mask=msk)
                        return cnt + csum[15]

                    return lax.cond(jnp.any(msk), matched, lambda c: c, cnt)

                cnt = lax.fori_loop(0, _EBLK // 16, scan_body, cnt)

                nchunks = cnt // _G

                def chunk_body(ci, _):
                    do_chunk(ci * _G)
                    return 0

                lax.fori_loop(0, nchunks, chunk_body, 0)

                @pl.when(nchunks > 0)
                def _():
                    base = nchunks * _G
                    for i in range(_G // 16):
                        mt_src[pl.ds(i * 16, 16)] = mt_src[pl.ds(base + i * 16, 16)]
                        mt_dst[pl.ds(i * 16, 16)] = mt_dst[pl.ds(base + i * 16, 16)]

                return cnt - nchunks * _G

            def block2(b2, cnt):
                cnt = do_block(b2 * 2, 0, cnt)
                cnt = do_block(b2 * 2 + 1, 1, cnt)
                return cnt

            cnt = lax.fori_loop(0, nblk // 2, block2, jnp.int32(0))

            for k in range(_G // 16):
                plsc.store_scatter(mt_dst, [cnt + k * 16 + iota], trash16)
                plsc.store_scatter(mt_src, [cnt + k * 16 + iota], zero16)

            @pl.when(cnt > 0)
            def _():
                do_chunk(0)

            pltpu.sync_copy(m_loc.at[pl.ds(0, _R * _OUT)],
                            out_hbm.at[pl.ds(rng * _R * _OUT, _R * _OUT)])

    return sc_fn


def _pass_a_body(f_ref, m_ref, a8_ref, p_ref, wg_ref, h_ref, s_ref, acc):
    i = pl.program_id(0)

    @pl.when(i == 0)
    def _():
        acc[...] = jnp.zeros_like(acc)

    fb = f_ref[...]
    xb = fb[:, 3:4]
    a8 = a8_ref[...]
    p = p_ref[...]
    vb = (xb * p[0:1, :] + p[1:2, :]
          - fb[:, 0:1] * a8[0:1, :]
          - fb[:, 1:2] * a8[1:2, :]
          - fb[:, 2:3] * a8[2:3, :])
    mb = m_ref[...]
    agg = jnp.where(jnp.isfinite(mb), mb + vb, 0.0)
    out = jnp.dot(agg, wg_ref[...], preferred_element_type=jnp.float32) + p[2:3, :]
    conv = xb + out
    h = jnp.where(conv > 0.0, conv, jnp.exp(conv) - 1.0)
    h_ref[...] = h
    rid = i * _BLK + lax.broadcasted_iota(jnp.int32, (_BLK, 1), 0)
    hm = jnp.where(rid < _N, h, 0.0)
    acc[0:1, :] = acc[0:1, :] + jnp.sum(hm, axis=0, keepdims=True)
    acc[1:2, :] = acc[1:2, :] + jnp.sum(hm * hm, axis=0, keepdims=True)

    @pl.when(i == _NB - 1)
    def _():
        s_ref[...] = acc[...]


def _pass_b_body(h_ref, s_ref, bn_ref, y_ref):
    s = s_ref[...]
    mean = s[0:1, :] / _N
    var = s[1:2, :] / _N - mean * mean
    scale = bn_ref[0:1, :] * lax.rsqrt(var + 1e-5)
    shift = bn_ref[1:2, :] - mean * scale
    y_ref[...] = h_ref[...] * scale + shift


def kernel(x, edge_index, edge_attr, Wh, bh, Wf, bf, Wg, bg, bn_w, bn_b):
    pos = edge_index
    src = edge_attr[0]
    dst = edge_attr[1]
    e = src.shape[0]
    nblk = -(-e // (2 * _EBLK)) * 2
    epad = nblk * _EBLK
    src_p = jnp.concatenate([src, jnp.zeros((epad - e,), jnp.int32)])
    dst_p = jnp.concatenate([dst, jnp.full((epad - e,), 1 << 30, jnp.int32)])
    f_tab = jnp.zeros((_NPAD, 16), jnp.float32)
    f_tab = f_tab.at[:_N, :3].set(pos).at[:_N, 3].set(x[:, 0])
    wft = Wf.T.astype(jnp.float32).reshape(-1)

    m_flat = _seg_max_sc(nblk)(src_p, dst_p, f_tab, wft)
    m = m_flat.reshape(_NPAD, _OUT)

    a3 = Wf[:, :3].T
    a8 = jnp.zeros((8, _OUT), jnp.float32).at[:3].set(a3)
    avec = Wh[:, 0] @ a3
    bvec = bh @ a3 + bf
    p = (jnp.zeros((8, _OUT), jnp.float32)
         .at[0].set(avec).at[1].set(bvec).at[2].set(bg))

    h, stats = pl.pallas_call(
        _pass_a_body,
        grid=(_NB,),
        in_specs=[
            pl.BlockSpec((_BLK, 16), lambda i: (i, 0)),
            pl.BlockSpec((_BLK, _OUT), lambda i: (i, 0)),
            pl.BlockSpec((8, _OUT), lambda i: (0, 0)),
            pl.BlockSpec((8, _OUT), lambda i: (0, 0)),
            pl.BlockSpec((_OUT, _OUT), lambda i: (0, 0)),
        ],
        out_specs=[
            pl.BlockSpec((_BLK, _OUT), lambda i: (i, 0)),
            pl.BlockSpec((8, _OUT), lambda i: (0, 0)),
        ],
        out_shape=[
            jax.ShapeDtypeStruct((_NPAD, _OUT), jnp.float32),
            jax.ShapeDtypeStruct((8, _OUT), jnp.float32),
        ],
        scratch_shapes=[pltpu.VMEM((8, _OUT), jnp.float32)],
    )(f_tab, m, a8, p, Wg.T.astype(jnp.float32))

    bn = jnp.zeros((8, _OUT), jnp.float32).at[0].set(bn_w).at[1].set(bn_b)

    y = pl.pallas_call(
        _pass_b_body,
        grid=(_NB,),
        in_specs=[
            pl.BlockSpec((_BLK, _OUT), lambda i: (i, 0)),
            pl.BlockSpec((8, _OUT), lambda i: (0, 0)),
            pl.BlockSpec((8, _OUT), lambda i: (0, 0)),
        ],
        out_specs=pl.BlockSpec((_BLK, _OUT), lambda i: (i, 0)),
        out_shape=jax.ShapeDtypeStruct((_NPAD, _OUT), jnp.float32),
    )(h, stats, bn)

    return y[:_N]


# speedup vs baseline: 5.8572x; 1.5515x over previous
import functools

import jax
import jax.numpy as jnp
from jax import lax
from jax.experimental import pallas as pl
from jax.experimental.pallas import tpu as pltpu
from jax.experimental.pallas import tpu_sc as plsc

_N = 50000
_OUT = 128
_R = 1568
_NPAD = 32 * _R
_EBLK = 2048
_G = 128
_PBUF = 2576
_MLOC = (_R + 1) * _OUT
_BLK = 512
_NB = _NPAD // _BLK

_mesh = plsc.VectorSubcoreMesh(core_axis_name="c", subcore_axis_name="s")


def _seg_max_sc(nblk):
    @functools.partial(
        pl.kernel,
        mesh=_mesh,
        compiler_params=pltpu.CompilerParams(
            needs_layout_passes=False, use_tc_tiling_on_sc=False),
        out_type=jax.ShapeDtypeStruct((_NPAD * _OUT,), jnp.bfloat16),
        scratch_types=[
            pltpu.VMEM((_MLOC,), jnp.bfloat16),
            pltpu.VMEM((_EBLK,), jnp.int32),
            pltpu.VMEM((_EBLK,), jnp.int32),
            pltpu.VMEM((_EBLK,), jnp.int32),
            pltpu.VMEM((_EBLK,), jnp.int32),
            pltpu.VMEM((_PBUF,), jnp.int32),
            pltpu.VMEM((_PBUF,), jnp.int32),
            pltpu.VMEM((_G, 16), jnp.float32),
            pltpu.VMEM((512,), jnp.float32),
            pltpu.SemaphoreType.DMA,
            pltpu.SemaphoreType.DMA,
            pltpu.SemaphoreType.DMA,
            pltpu.SemaphoreType.DMA,
            pltpu.SemaphoreType.DMA,
        ],
    )
    def sc_fn(src_hbm, dst_hbm, ftab_hbm, wft_hbm, out_hbm,
              m_loc, dbuf0, dbuf1, sbuf0, sbuf1, mt_src, mt_dst, rows, wf_v,
              sd0, sd1, ss0, ss1, gsem):
        wid = lax.axis_index("s") * 2 + lax.axis_index("c")
        pltpu.sync_copy(wft_hbm, wf_v)
        ws = [wf_v[pl.ds(t * 16, 16)] for t in range(32)]
        iota = lax.iota(jnp.int32, 16)
        neg = jnp.full((32,), -jnp.inf, jnp.bfloat16)
        zero16 = jnp.zeros((16,), jnp.int32)
        trash16 = jnp.full((16,), _R, jnp.int32)
        sd = [sd0, sd1]
        ss = [ss0, ss1]
        dbuf = [dbuf0, dbuf1]
        sbuf = [sbuf0, sbuf1]

        def do_chunk(off):
            pltpu.async_copy(ftab_hbm.at[mt_src.at[pl.ds(off, _G)]],
                             rows, gsem).wait()

            def group_body(g, _):
                dl16 = mt_dst[pl.ds(off + g * 16, 16)]
                for lane in range(16):
                    rv = rows[g * 16 + lane, :]
                    fx = rv[0]
                    fy = rv[1]
                    fz = rv[2]
                    fw = rv[3]
                    base = dl16[lane] * _OUT
                    for p in range(4):
                        acc_a = (ws[p * 2] * fx + ws[8 + p * 2] * fy
                                 + ws[16 + p * 2] * fz + ws[24 + p * 2] * fw)
                        acc_b = (ws[p * 2 + 1] * fx + ws[8 + p * 2 + 1] * fy
                                 + ws[16 + p * 2 + 1] * fz
                                 + ws[24 + p * 2 + 1] * fw)
                        packed = plsc.pack(acc_a, acc_b,
                                           format=plsc.PackFormat.INTERLEAVED)
                        sl = pl.ds(base + p * 32, 32)
                        m_loc[sl] = jnp.maximum(m_loc[sl], packed)
                return 0

            lax.fori_loop(0, _G // 16, group_body, 0)

        if True:
            rng = wid
            lo = rng * _R
            hi = lo + _R

            def init_body(i, _):
                m_loc[pl.ds(i * 32, 32)] = neg
                return 0

            lax.fori_loop(0, _MLOC // 32, init_body, 0)

            pltpu.async_copy(dst_hbm.at[pl.ds(0, _EBLK)], dbuf[0], sd[0])
            pltpu.async_copy(src_hbm.at[pl.ds(0, _EBLK)], sbuf[0], ss[0])

            def do_block(b, slot, cnt, lo=lo, hi=hi):
                pltpu.make_async_copy(dst_hbm.at[pl.ds(0, _EBLK)],
                                      dbuf[slot], sd[slot]).wait()
                pltpu.make_async_copy(src_hbm.at[pl.ds(0, _EBLK)],
                                      sbuf[slot], ss[slot]).wait()

                @pl.when(b + 1 < nblk)
                def _():
                    nb = (b + 1) * _EBLK
                    pltpu.async_copy(dst_hbm.at[pl.ds(nb, _EBLK)],
                                     dbuf[1 - slot], sd[1 - slot])
                    pltpu.async_copy(src_hbm.at[pl.ds(nb, _EBLK)],
                                     sbuf[1 - slot], ss[1 - slot])

                dref = dbuf[slot]
                sref = sbuf[slot]

                def scan_body(i, cnt):
                    d = dref[pl.ds(i * 16, 16)]
                    msk = (d >= lo) & (d < hi)

                    def matched(cnt):
                        s = sref[pl.ds(i * 16, 16)]
                        mi = msk.astype(jnp.int32)
                        csum = plsc.cumsum(mi)
                        pos = cnt + csum - mi
                        plsc.store_scatter(mt_dst, [pos], d - lo, mask=msk)
                        plsc.store_scatter(mt_src, [pos], s, mask=msk)
                        return cnt + csum[15]

                    return lax.cond(jnp.any(msk), matched, lambda c: c, cnt)

                cnt = lax.fori_loop(0, _EBLK // 16, scan_body, cnt)

                nchunks = cnt // _G

                def chunk_body(ci, _):
                    do_chunk(ci * _G)
                    return 0

                lax.fori_loop(0, nchunks, chunk_body, 0)

                @pl.when(nchunks > 0)
                def _():
                    base = nchunks * _G
                    for i in range(_G // 16):
                        mt_src[pl.ds(i * 16, 16)] = mt_src[pl.ds(base + i * 16, 16)]
                        mt_dst[pl.ds(i * 16, 16)] = mt_dst[pl.ds(base + i * 16, 16)]

                return cnt - nchunks * _G

            def block2(b2, cnt):
                cnt = do_block(b2 * 2, 0, cnt)
                cnt = do_block(b2 * 2 + 1, 1, cnt)
                return cnt

            cnt = lax.fori_loop(0, nblk // 2, block2, jnp.int32(0))

            for k in range(_G // 16):
                plsc.store_scatter(mt_dst, [cnt + k * 16 + iota], trash16)
                plsc.store_scatter(mt_src, [cnt + k * 16 + iota], zero16)

            @pl.when(cnt > 0)
            def _():
                do_chunk(0)

            pltpu.sync_copy(m_loc.at[pl.ds(0, _R * _OUT)],
                            out_hbm.at[pl.ds(rng * _R * _OUT, _R * _OUT)])

    return sc_fn


def _pass_a_body(f_ref, m_ref, a8_ref, p_ref, wg_ref, h_ref, s_ref, acc):
    i = pl.program_id(0)

    @pl.when(i == 0)
    def _():
        acc[...] = jnp.zeros_like(acc)

    fb = f_ref[...]
    xb = fb[:, 3:4]
    a8 = a8_ref[...]
    p = p_ref[...]
    vb = (xb * p[0:1, :] + p[1:2, :]
          - fb[:, 0:1] * a8[0:1, :]
          - fb[:, 1:2] * a8[1:2, :]
          - fb[:, 2:3] * a8[2:3, :])
    mb = m_ref[...].astype(jnp.float32)
    agg = jnp.where(jnp.isfinite(mb), mb + vb, 0.0)
    out = jnp.dot(agg, wg_ref[...], preferred_element_type=jnp.float32) + p[2:3, :]
    conv = xb + out
    h = jnp.where(conv > 0.0, conv, jnp.exp(conv) - 1.0)
    h_ref[...] = h
    rid = i * _BLK + lax.broadcasted_iota(jnp.int32, (_BLK, 1), 0)
    hm = jnp.where(rid < _N, h, 0.0)
    acc[0:1, :] = acc[0:1, :] + jnp.sum(hm, axis=0, keepdims=True)
    acc[1:2, :] = acc[1:2, :] + jnp.sum(hm * hm, axis=0, keepdims=True)

    @pl.when(i == _NB - 1)
    def _():
        s_ref[...] = acc[...]


def _pass_b_body(h_ref, s_ref, bn_ref, y_ref):
    s = s_ref[...]
    mean = s[0:1, :] / _N
    var = s[1:2, :] / _N - mean * mean
    scale = bn_ref[0:1, :] * lax.rsqrt(var + 1e-5)
    shift = bn_ref[1:2, :] - mean * scale
    y_ref[...] = h_ref[...] * scale + shift


def kernel(x, edge_index, edge_attr, Wh, bh, Wf, bf, Wg, bg, bn_w, bn_b):
    pos = edge_index
    src = edge_attr[0]
    dst = edge_attr[1]
    e = src.shape[0]
    nblk = -(-e // (2 * _EBLK)) * 2
    epad = nblk * _EBLK
    src_p = jnp.concatenate([src, jnp.zeros((epad - e,), jnp.int32)])
    dst_p = jnp.concatenate([dst, jnp.full((epad - e,), 1 << 30, jnp.int32)])
    f_tab = jnp.zeros((_NPAD, 16), jnp.float32)
    f_tab = f_tab.at[:_N, :3].set(pos).at[:_N, 3].set(x[:, 0])
    wft4 = Wf.T.astype(jnp.float32).reshape(4, 4, 32)
    wft = jnp.stack([wft4[:, :, 0::2], wft4[:, :, 1::2]], axis=2).reshape(-1)

    m_flat = _seg_max_sc(nblk)(src_p, dst_p, f_tab, wft)
    m = m_flat.reshape(_NPAD, _OUT)

    a3 = Wf[:, :3].T
    a8 = jnp.zeros((8, _OUT), jnp.float32).at[:3].set(a3)
    avec = Wh[:, 0] @ a3
    bvec = bh @ a3 + bf
    p = (jnp.zeros((8, _OUT), jnp.float32)
         .at[0].set(avec).at[1].set(bvec).at[2].set(bg))

    h, stats = pl.pallas_call(
        _pass_a_body,
        grid=(_NB,),
        in_specs=[
            pl.BlockSpec((_BLK, 16), lambda i: (i, 0)),
            pl.BlockSpec((_BLK, _OUT), lambda i: (i, 0)),
            pl.BlockSpec((8, _OUT), lambda i: (0, 0)),
            pl.BlockSpec((8, _OUT), lambda i: (0, 0)),
            pl.BlockSpec((_OUT, _OUT), lambda i: (0, 0)),
        ],
        out_specs=[
            pl.BlockSpec((_BLK, _OUT), lambda i: (i, 0)),
            pl.BlockSpec((8, _OUT), lambda i: (0, 0)),
        ],
        out_shape=[
            jax.ShapeDtypeStruct((_NPAD, _OUT), jnp.float32),
            jax.ShapeDtypeStruct((8, _OUT), jnp.float32),
        ],
        scratch_shapes=[pltpu.VMEM((8, _OUT), jnp.float32)],
    )(f_tab, m, a8, p, Wg.T.astype(jnp.float32))

    bn = jnp.zeros((8, _OUT), jnp.float32).at[0].set(bn_w).at[1].set(bn_b)

    y = pl.pallas_call(
        _pass_b_body,
        grid=(_NB,),
        in_specs=[
            pl.BlockSpec((_BLK, _OUT), lambda i: (i, 0)),
            pl.BlockSpec((8, _OUT), lambda i: (0, 0)),
            pl.BlockSpec((8, _OUT), lambda i: (0, 0)),
        ],
        out_specs=pl.BlockSpec((_BLK, _OUT), lambda i: (i, 0)),
        out_shape=jax.ShapeDtypeStruct((_NPAD, _OUT), jnp.float32),
    )(h, stats, bn)

    return y[:_N]


# speedup vs baseline: 8.2646x; 1.4110x over previous
import functools

import jax
import jax.numpy as jnp
from jax import lax
from jax.experimental import pallas as pl
from jax.experimental.pallas import tpu as pltpu
from jax.experimental.pallas import tpu_sc as plsc

_N = 50000
_OUT = 128
_R = 1568
_NPAD = 32 * _R
_EBLK = 2048
_G = 128
_PBUF = 2576
_MLOC = (_R + 1) * _OUT
_BLK = 512
_NB = _NPAD // _BLK

_mesh = plsc.VectorSubcoreMesh(core_axis_name="c", subcore_axis_name="s")


def _seg_max_sc(nblk):
    @functools.partial(
        pl.kernel,
        mesh=_mesh,
        compiler_params=pltpu.CompilerParams(
            needs_layout_passes=False, use_tc_tiling_on_sc=False),
        out_type=jax.ShapeDtypeStruct((_NPAD * _OUT,), jnp.bfloat16),
        scratch_types=[
            pltpu.VMEM((_MLOC,), jnp.bfloat16),
            pltpu.VMEM((_EBLK,), jnp.int32),
            pltpu.VMEM((_EBLK,), jnp.int32),
            pltpu.VMEM((_EBLK,), jnp.int32),
            pltpu.VMEM((_EBLK,), jnp.int32),
            pltpu.VMEM((_PBUF,), jnp.int32),
            pltpu.VMEM((_PBUF,), jnp.int32),
            pltpu.VMEM((_G, 16), jnp.float32),
            pltpu.VMEM((512,), jnp.float32),
            pltpu.SemaphoreType.DMA,
            pltpu.SemaphoreType.DMA,
            pltpu.SemaphoreType.DMA,
            pltpu.SemaphoreType.DMA,
            pltpu.SemaphoreType.DMA,
        ],
    )
    def sc_fn(src_hbm, dst_hbm, ftab_hbm, wft_hbm, out_hbm,
              m_loc, dbuf0, dbuf1, sbuf0, sbuf1, mt_src, mt_dst, rows, wf_v,
              sd0, sd1, ss0, ss1, gsem):
        wid = lax.axis_index("s") * 2 + lax.axis_index("c")
        pltpu.sync_copy(wft_hbm, wf_v)
        ws = [wf_v[pl.ds(t * 16, 16)] for t in range(32)]
        iota = lax.iota(jnp.int32, 16)
        neg = jnp.full((32,), -jnp.inf, jnp.bfloat16)
        zero16 = jnp.zeros((16,), jnp.int32)
        trash16 = jnp.full((16,), _R, jnp.int32)
        sd = [sd0, sd1]
        ss = [ss0, ss1]
        dbuf = [dbuf0, dbuf1]
        sbuf = [sbuf0, sbuf1]

        def do_chunk(off):
            pltpu.async_copy(ftab_hbm.at[mt_src.at[pl.ds(off, _G)]],
                             rows, gsem).wait()

            def group_body(g, _):
                dl16 = mt_dst[pl.ds(off + g * 16, 16)]
                for lane in range(16):
                    rv = rows[g * 16 + lane, :]
                    fx = rv[0]
                    fy = rv[1]
                    fz = rv[2]
                    fw = rv[3]
                    base = dl16[lane] * _OUT
                    for p in range(4):
                        acc_a = (ws[p * 2] * fx + ws[8 + p * 2] * fy
                                 + ws[16 + p * 2] * fz + ws[24 + p * 2] * fw)
                        acc_b = (ws[p * 2 + 1] * fx + ws[8 + p * 2 + 1] * fy
                                 + ws[16 + p * 2 + 1] * fz
                                 + ws[24 + p * 2 + 1] * fw)
                        packed = plsc.pack(acc_a, acc_b,
                                           format=plsc.PackFormat.INTERLEAVED)
                        sl = pl.ds(base + p * 32, 32)
                        m_loc[sl] = jnp.maximum(m_loc[sl], packed)
                return 0

            lax.fori_loop(0, _G // 16, group_body, 0)

        if True:
            rng = wid
            lo = rng * _R
            hi = lo + _R

            def init_body(i, _):
                m_loc[pl.ds(i * 32, 32)] = neg
                return 0

            lax.fori_loop(0, _MLOC // 32, init_body, 0)

            pltpu.async_copy(dst_hbm.at[pl.ds(0, _EBLK)], dbuf[0], sd[0])
            pltpu.async_copy(src_hbm.at[pl.ds(0, _EBLK)], sbuf[0], ss[0])

            def do_block(b, slot, cnt, lo=lo, hi=hi):
                pltpu.make_async_copy(dst_hbm.at[pl.ds(0, _EBLK)],
                                      dbuf[slot], sd[slot]).wait()
                pltpu.make_async_copy(src_hbm.at[pl.ds(0, _EBLK)],
                                      sbuf[slot], ss[slot]).wait()

                @pl.when(b + 1 < nblk)
                def _():
                    nb = (b + 1) * _EBLK
                    pltpu.async_copy(dst_hbm.at[pl.ds(nb, _EBLK)],
                                     dbuf[1 - slot], sd[1 - slot])
                    pltpu.async_copy(src_hbm.at[pl.ds(nb, _EBLK)],
                                     sbuf[1 - slot], ss[1 - slot])

                dref = dbuf[slot]
                sref = sbuf[slot]

                def scan_body(i, cnt):
                    d = dref[pl.ds(i * 16, 16)]
                    s = sref[pl.ds(i * 16, 16)]
                    dl = d - lo
                    msk = dl.astype(jnp.uint32) < jnp.uint32(_R)
                    mi = msk.astype(jnp.int32)
                    csum = plsc.cumsum(mi)
                    pos = cnt + csum - mi
                    plsc.store_scatter(mt_dst, [pos], dl, mask=msk)
                    plsc.store_scatter(mt_src, [pos], s, mask=msk)
                    pc = plsc.all_reduce_population_count(msk)
                    return cnt + pc[0]

                cnt = lax.fori_loop(0, _EBLK // 16, scan_body, cnt,
                                    unroll=4)

                nchunks = cnt // _G

                def chunk_body(ci, _):
                    do_chunk(ci * _G)
                    return 0

                lax.fori_loop(0, nchunks, chunk_body, 0)

                @pl.when(nchunks > 0)
                def _():
                    base = nchunks * _G
                    for i in range(_G // 16):
                        mt_src[pl.ds(i * 16, 16)] = mt_src[pl.ds(base + i * 16, 16)]
                        mt_dst[pl.ds(i * 16, 16)] = mt_dst[pl.ds(base + i * 16, 16)]

                return cnt - nchunks * _G

            def block2(b2, cnt):
                cnt = do_block(b2 * 2, 0, cnt)
                cnt = do_block(b2 * 2 + 1, 1, cnt)
                return cnt

            cnt = lax.fori_loop(0, nblk // 2, block2, jnp.int32(0))

            for k in range(_G // 16):
                plsc.store_scatter(mt_dst, [cnt + k * 16 + iota], trash16)
                plsc.store_scatter(mt_src, [cnt + k * 16 + iota], zero16)

            @pl.when(cnt > 0)
            def _():
                do_chunk(0)

            pltpu.sync_copy(m_loc.at[pl.ds(0, _R * _OUT)],
                            out_hbm.at[pl.ds(rng * _R * _OUT, _R * _OUT)])

    return sc_fn


def _pass_a_body(f_ref, m_ref, a8_ref, p_ref, wg_ref, h_ref, s_ref, acc):
    i = pl.program_id(0)

    @pl.when(i == 0)
    def _():
        acc[...] = jnp.zeros_like(acc)

    fb = f_ref[...]
    xb = fb[:, 3:4]
    a8 = a8_ref[...]
    p = p_ref[...]
    vb = (xb * p[0:1, :] + p[1:2, :]
          - fb[:, 0:1] * a8[0:1, :]
          - fb[:, 1:2] * a8[1:2, :]
          - fb[:, 2:3] * a8[2:3, :])
    mb = m_ref[...].astype(jnp.float32)
    agg = jnp.where(jnp.isfinite(mb), mb + vb, 0.0)
    out = jnp.dot(agg, wg_ref[...], preferred_element_type=jnp.float32) + p[2:3, :]
    conv = xb + out
    h = jnp.where(conv > 0.0, conv, jnp.exp(conv) - 1.0)
    h_ref[...] = h
    rid = i * _BLK + lax.broadcasted_iota(jnp.int32, (_BLK, 1), 0)
    hm = jnp.where(rid < _N, h, 0.0)
    acc[0:1, :] = acc[0:1, :] + jnp.sum(hm, axis=0, keepdims=True)
    acc[1:2, :] = acc[1:2, :] + jnp.sum(hm * hm, axis=0, keepdims=True)

    @pl.when(i == _NB - 1)
    def _():
        s_ref[...] = acc[...]


def _pass_b_body(h_ref, s_ref, bn_ref, y_ref):
    s = s_ref[...]
    mean = s[0:1, :] / _N
    var = s[1:2, :] / _N - mean * mean
    scale = bn_ref[0:1, :] * lax.rsqrt(var + 1e-5)
    shift = bn_ref[1:2, :] - mean * scale
    y_ref[...] = h_ref[...] * scale + shift


def kernel(x, edge_index, edge_attr, Wh, bh, Wf, bf, Wg, bg, bn_w, bn_b):
    pos = edge_index
    src = edge_attr[0]
    dst = edge_attr[1]
    e = src.shape[0]
    nblk = -(-e // (2 * _EBLK)) * 2
    epad = nblk * _EBLK
    src_p = jnp.concatenate([src, jnp.zeros((epad - e,), jnp.int32)])
    dst_p = jnp.concatenate([dst, jnp.full((epad - e,), 1 << 30, jnp.int32)])
    f_tab = jnp.zeros((_NPAD, 16), jnp.float32)
    f_tab = f_tab.at[:_N, :3].set(pos).at[:_N, 3].set(x[:, 0])
    wft4 = Wf.T.astype(jnp.float32).reshape(4, 4, 32)
    wft = jnp.stack([wft4[:, :, 0::2], wft4[:, :, 1::2]], axis=2).reshape(-1)

    m_flat = _seg_max_sc(nblk)(src_p, dst_p, f_tab, wft)
    m = m_flat.reshape(_NPAD, _OUT)

    a3 = Wf[:, :3].T
    a8 = jnp.zeros((8, _OUT), jnp.float32).at[:3].set(a3)
    avec = Wh[:, 0] @ a3
    bvec = bh @ a3 + bf
    p = (jnp.zeros((8, _OUT), jnp.float32)
         .at[0].set(avec).at[1].set(bvec).at[2].set(bg))

    h, stats = pl.pallas_call(
        _pass_a_body,
        grid=(_NB,),
        in_specs=[
            pl.BlockSpec((_BLK, 16), lambda i: (i, 0)),
            pl.BlockSpec((_BLK, _OUT), lambda i: (i, 0)),
            pl.BlockSpec((8, _OUT), lambda i: (0, 0)),
            pl.BlockSpec((8, _OUT), lambda i: (0, 0)),
            pl.BlockSpec((_OUT, _OUT), lambda i: (0, 0)),
        ],
        out_specs=[
            pl.BlockSpec((_BLK, _OUT), lambda i: (i, 0)),
            pl.BlockSpec((8, _OUT), lambda i: (0, 0)),
        ],
        out_shape=[
            jax.ShapeDtypeStruct((_NPAD, _OUT), jnp.float32),
            jax.ShapeDtypeStruct((8, _OUT), jnp.float32),
        ],
        scratch_shapes=[pltpu.VMEM((8, _OUT), jnp.float32)],
    )(f_tab, m, a8, p, Wg.T.astype(jnp.float32))

    bn = jnp.zeros((8, _OUT), jnp.float32).at[0].set(bn_w).at[1].set(bn_b)

    y = pl.pallas_call(
        _pass_b_body,
        grid=(_NB,),
        in_specs=[
            pl.BlockSpec((_BLK, _OUT), lambda i: (i, 0)),
            pl.BlockSpec((8, _OUT), lambda i: (0, 0)),
            pl.BlockSpec((8, _OUT), lambda i: (0, 0)),
        ],
        out_specs=pl.BlockSpec((_BLK, _OUT), lambda i: (i, 0)),
        out_shape=jax.ShapeDtypeStruct((_NPAD, _OUT), jnp.float32),
    )(h, stats, bn)

    return y[:_N]


# speedup vs baseline: 8.5290x; 1.0320x over previous
import functools

import jax
import jax.numpy as jnp
from jax import lax
from jax.experimental import pallas as pl
from jax.experimental.pallas import tpu as pltpu
from jax.experimental.pallas import tpu_sc as plsc

_N = 50000
_OUT = 128
_R = 1568
_NPAD = 32 * _R
_EBLK = 2048
_G = 512
_GSUB = 128
_PBUF = 2592
_MLOC = (_R + 1) * _OUT
_BLK = 512
_NB = _NPAD // _BLK

_mesh = plsc.VectorSubcoreMesh(core_axis_name="c", subcore_axis_name="s")


def _seg_max_sc(nblk):
    @functools.partial(
        pl.kernel,
        mesh=_mesh,
        compiler_params=pltpu.CompilerParams(
            needs_layout_passes=False, use_tc_tiling_on_sc=False),
        out_type=jax.ShapeDtypeStruct((_NPAD * _OUT,), jnp.bfloat16),
        scratch_types=[
            pltpu.VMEM((_MLOC,), jnp.bfloat16),
            pltpu.VMEM((_EBLK,), jnp.int32),
            pltpu.VMEM((_EBLK,), jnp.int32),
            pltpu.VMEM((_EBLK,), jnp.int32),
            pltpu.VMEM((_EBLK,), jnp.int32),
            pltpu.VMEM((_PBUF,), jnp.int32),
            pltpu.VMEM((_PBUF,), jnp.int32),
            pltpu.VMEM((_G, 16), jnp.float32),

            pltpu.VMEM((512,), jnp.float32),
            pltpu.SemaphoreType.DMA,
            pltpu.SemaphoreType.DMA,
            pltpu.SemaphoreType.DMA,
            pltpu.SemaphoreType.DMA,
            pltpu.SemaphoreType.DMA,
        ],
    )
    def sc_fn(src_hbm, dst_hbm, ftab_hbm, wft_hbm, out_hbm,
              m_loc, dbuf0, dbuf1, sbuf0, sbuf1, mt_src, mt_dst, rows, wf_v,
              sd0, sd1, ss0, ss1, gsem):
        wid = lax.axis_index("s") * 2 + lax.axis_index("c")
        pltpu.sync_copy(wft_hbm, wf_v)
        ws = [wf_v[pl.ds(t * 16, 16)] for t in range(32)]
        iota = lax.iota(jnp.int32, 16)
        neg = jnp.full((32,), -jnp.inf, jnp.bfloat16)
        zero16 = jnp.zeros((16,), jnp.int32)
        trash16 = jnp.full((16,), _R, jnp.int32)
        sd = [sd0, sd1]
        ss = [ss0, ss1]
        dbuf = [dbuf0, dbuf1]
        sbuf = [sbuf0, sbuf1]

        def do_chunk(off):
            cps = [pltpu.async_copy(
                       ftab_hbm.at[mt_src.at[pl.ds(off + q * _GSUB, _GSUB)]],
                       rows.at[pl.ds(q * _GSUB, _GSUB)], gsem)
                   for q in range(_G // _GSUB)]
            for cp in cps:
                cp.wait()

            def group_body(g, _):
                dl16 = mt_dst[pl.ds(off + g * 16, 16)]
                for lane in range(16):
                    rv = rows[g * 16 + lane, :]
                    fx = rv[0]
                    fy = rv[1]
                    fz = rv[2]
                    fw = rv[3]
                    base = dl16[lane] * _OUT
                    for p in range(4):
                        acc_a = (ws[p * 2] * fx + ws[8 + p * 2] * fy
                                 + ws[16 + p * 2] * fz + ws[24 + p * 2] * fw)
                        acc_b = (ws[p * 2 + 1] * fx + ws[8 + p * 2 + 1] * fy
                                 + ws[16 + p * 2 + 1] * fz
                                 + ws[24 + p * 2 + 1] * fw)
                        packed = plsc.pack(acc_a, acc_b,
                                           format=plsc.PackFormat.INTERLEAVED)
                        sl = pl.ds(base + p * 32, 32)
                        m_loc[sl] = jnp.maximum(m_loc[sl], packed)
                return 0

            lax.fori_loop(0, _G // 16, group_body, 0)

        if True:
            rng = wid
            lo = rng * _R
            hi = lo + _R

            def init_body(i, _):
                m_loc[pl.ds(i * 32, 32)] = neg
                return 0

            lax.fori_loop(0, _MLOC // 32, init_body, 0)

            pltpu.async_copy(dst_hbm.at[pl.ds(0, _EBLK)], dbuf[0], sd[0])
            pltpu.async_copy(src_hbm.at[pl.ds(0, _EBLK)], sbuf[0], ss[0])

            def do_block(b, slot, cnt, lo=lo, hi=hi):
                pltpu.make_async_copy(dst_hbm.at[pl.ds(0, _EBLK)],
                                      dbuf[slot], sd[slot]).wait()
                pltpu.make_async_copy(src_hbm.at[pl.ds(0, _EBLK)],
                                      sbuf[slot], ss[slot]).wait()

                @pl.when(b + 1 < nblk)
                def _():
                    nb = (b + 1) * _EBLK
                    pltpu.async_copy(dst_hbm.at[pl.ds(nb, _EBLK)],
                                     dbuf[1 - slot], sd[1 - slot])
                    pltpu.async_copy(src_hbm.at[pl.ds(nb, _EBLK)],
                                     sbuf[1 - slot], ss[1 - slot])

                dref = dbuf[slot]
                sref = sbuf[slot]

                def scan_body(i, cnt):
                    d = dref[pl.ds(i * 16, 16)]
                    s = sref[pl.ds(i * 16, 16)]
                    dl = d - lo
                    msk = dl.astype(jnp.uint32) < jnp.uint32(_R)
                    mi = msk.astype(jnp.int32)
                    csum = plsc.cumsum(mi)
                    pos = cnt + csum - mi
                    plsc.store_scatter(mt_dst, [pos], dl, mask=msk)
                    plsc.store_scatter(mt_src, [pos], s, mask=msk)
                    pc = plsc.all_reduce_population_count(msk)
                    return cnt + pc[0]

                cnt = lax.fori_loop(0, _EBLK // 16, scan_body, cnt,
                                    unroll=4)

                nchunks = cnt // _G

                def chunk_body(ci, _):
                    do_chunk(ci * _G)
                    return 0

                lax.fori_loop(0, nchunks, chunk_body, 0)

                @pl.when(nchunks > 0)
                def _():
                    base = nchunks * _G
                    for i in range(_G // 16):
                        mt_src[pl.ds(i * 16, 16)] = mt_src[pl.ds(base + i * 16, 16)]
                        mt_dst[pl.ds(i * 16, 16)] = mt_dst[pl.ds(base + i * 16, 16)]

                return cnt - nchunks * _G

            def block2(b2, cnt):
                cnt = do_block(b2 * 2, 0, cnt)
                cnt = do_block(b2 * 2 + 1, 1, cnt)
                return cnt

            cnt = lax.fori_loop(0, nblk // 2, block2, jnp.int32(0))

            for k in range(_G // 16):
                plsc.store_scatter(mt_dst, [cnt + k * 16 + iota], trash16)
                plsc.store_scatter(mt_src, [cnt + k * 16 + iota], zero16)

            @pl.when(cnt > 0)
            def _():
                do_chunk(0)

            pltpu.sync_copy(m_loc.at[pl.ds(0, _R * _OUT)],
                            out_hbm.at[pl.ds(rng * _R * _OUT, _R * _OUT)])

    return sc_fn


def _pass_a_body(f_ref, m_ref, a8_ref, p_ref, wg_ref, h_ref, s_ref, acc):
    i = pl.program_id(0)

    @pl.when(i == 0)
    def _():
        acc[...] = jnp.zeros_like(acc)

    fb = f_ref[...]
    xb = fb[:, 3:4]
    a8 = a8_ref[...]
    p = p_ref[...]
    vb = (xb * p[0:1, :] + p[1:2, :]
          - fb[:, 0:1] * a8[0:1, :]
          - fb[:, 1:2] * a8[1:2, :]
          - fb[:, 2:3] * a8[2:3, :])
    mb = m_ref[...].astype(jnp.float32)
    agg = jnp.where(jnp.isfinite(mb), mb + vb, 0.0)
    out = jnp.dot(agg, wg_ref[...], preferred_element_type=jnp.float32) + p[2:3, :]
    conv = xb + out
    h = jnp.where(conv > 0.0, conv, jnp.exp(conv) - 1.0)
    h_ref[...] = h
    rid = i * _BLK + lax.broadcasted_iota(jnp.int32, (_BLK, 1), 0)
    hm = jnp.where(rid < _N, h, 0.0)
    acc[0:1, :] = acc[0:1, :] + jnp.sum(hm, axis=0, keepdims=True)
    acc[1:2, :] = acc[1:2, :] + jnp.sum(hm * hm, axis=0, keepdims=True)

    @pl.when(i == _NB - 1)
    def _():
        s_ref[...] = acc[...]


def _pass_b_body(h_ref, s_ref, bn_ref, y_ref):
    s = s_ref[...]
    mean = s[0:1, :] / _N
    var = s[1:2, :] / _N - mean * mean
    scale = bn_ref[0:1, :] * lax.rsqrt(var + 1e-5)
    shift = bn_ref[1:2, :] - mean * scale
    y_ref[...] = h_ref[...] * scale + shift


def kernel(x, edge_index, edge_attr, Wh, bh, Wf, bf, Wg, bg, bn_w, bn_b):
    pos = edge_index
    src = edge_attr[0]
    dst = edge_attr[1]
    e = src.shape[0]
    nblk = -(-e // (2 * _EBLK)) * 2
    epad = nblk * _EBLK
    src_p = jnp.concatenate([src, jnp.zeros((epad - e,), jnp.int32)])
    dst_p = jnp.concatenate([dst, jnp.full((epad - e,), 1 << 30, jnp.int32)])
    f_tab = jnp.zeros((_NPAD, 16), jnp.float32)
    f_tab = f_tab.at[:_N, :3].set(pos).at[:_N, 3].set(x[:, 0])
    wft4 = Wf.T.astype(jnp.float32).reshape(4, 4, 32)
    wft = jnp.stack([wft4[:, :, 0::2], wft4[:, :, 1::2]], axis=2).reshape(-1)

    m_flat = _seg_max_sc(nblk)(src_p, dst_p, f_tab, wft)
    m = m_flat.reshape(_NPAD, _OUT)

    a3 = Wf[:, :3].T
    a8 = jnp.zeros((8, _OUT), jnp.float32).at[:3].set(a3)
    avec = Wh[:, 0] @ a3
    bvec = bh @ a3 + bf
    p = (jnp.zeros((8, _OUT), jnp.float32)
         .at[0].set(avec).at[1].set(bvec).at[2].set(bg))

    h, stats = pl.pallas_call(
        _pass_a_body,
        grid=(_NB,),
        in_specs=[
            pl.BlockSpec((_BLK, 16), lambda i: (i, 0)),
            pl.BlockSpec((_BLK, _OUT), lambda i: (i, 0)),
            pl.BlockSpec((8, _OUT), lambda i: (0, 0)),
            pl.BlockSpec((8, _OUT), lambda i: (0, 0)),
            pl.BlockSpec((_OUT, _OUT), lambda i: (0, 0)),
        ],
        out_specs=[
            pl.BlockSpec((_BLK, _OUT), lambda i: (i, 0)),
            pl.BlockSpec((8, _OUT), lambda i: (0, 0)),
        ],
        out_shape=[
            jax.ShapeDtypeStruct((_NPAD, _OUT), jnp.float32),
            jax.ShapeDtypeStruct((8, _OUT), jnp.float32),
        ],
        scratch_shapes=[pltpu.VMEM((8, _OUT), jnp.float32)],
    )(f_tab, m, a8, p, Wg.T.astype(jnp.float32))

    bn = jnp.zeros((8, _OUT), jnp.float32).at[0].set(bn_w).at[1].set(bn_b)

    y = pl.pallas_call(
        _pass_b_body,
        grid=(_NB,),
        in_specs=[
            pl.BlockSpec((_BLK, _OUT), lambda i: (i, 0)),
            pl.BlockSpec((8, _OUT), lambda i: (0, 0)),
            pl.BlockSpec((8, _OUT), lambda i: (0, 0)),
        ],
        out_specs=pl.BlockSpec((_BLK, _OUT), lambda i: (i, 0)),
        out_shape=jax.ShapeDtypeStruct((_NPAD, _OUT), jnp.float32),
    )(h, stats, bn)

    return y[:_N]


# speedup vs baseline: 8.6082x; 1.0093x over previous
import functools

import jax
import jax.numpy as jnp
from jax import lax
from jax.experimental import pallas as pl
from jax.experimental.pallas import tpu as pltpu
from jax.experimental.pallas import tpu_sc as plsc

_N = 50000
_OUT = 128
_R = 1568
_NPAD = 32 * _R
_EBLK = 2048
_G = 512
_GSUB = 128
_PBUF = 2592
_MLOC = (_R + 1) * _OUT
_BLK = 512
_NB = _NPAD // _BLK

_mesh = plsc.VectorSubcoreMesh(core_axis_name="c", subcore_axis_name="s")


def _seg_max_sc(nblk):
    @functools.partial(
        pl.kernel,
        mesh=_mesh,
        compiler_params=pltpu.CompilerParams(
            needs_layout_passes=False, use_tc_tiling_on_sc=False),
        out_type=jax.ShapeDtypeStruct((_NPAD * _OUT,), jnp.bfloat16),
        scratch_types=[
            pltpu.VMEM((_MLOC,), jnp.bfloat16),
            pltpu.VMEM((_EBLK,), jnp.int32),
            pltpu.VMEM((_EBLK,), jnp.int32),
            pltpu.VMEM((_EBLK,), jnp.int32),
            pltpu.VMEM((_EBLK,), jnp.int32),
            pltpu.VMEM((_PBUF,), jnp.int32),
            pltpu.VMEM((_PBUF,), jnp.int32),
            pltpu.VMEM((_G, 16), jnp.float32),

            pltpu.VMEM((512,), jnp.float32),
            pltpu.SemaphoreType.DMA,
            pltpu.SemaphoreType.DMA,
            pltpu.SemaphoreType.DMA,
            pltpu.SemaphoreType.DMA,
            pltpu.SemaphoreType.DMA,
        ],
    )
    def sc_fn(src_hbm, dst_hbm, ftab_hbm, wft_hbm, out_hbm,
              m_loc, dbuf0, dbuf1, sbuf0, sbuf1, mt_src, mt_dst, rows, wf_v,
              sd0, sd1, ss0, ss1, gsem):
        wid = lax.axis_index("s") * 2 + lax.axis_index("c")
        pltpu.sync_copy(wft_hbm, wf_v)
        ws = [wf_v[pl.ds(t * 16, 16)] for t in range(32)]
        iota = lax.iota(jnp.int32, 16)
        neg = jnp.full((32,), -jnp.inf, jnp.bfloat16)
        zero16 = jnp.zeros((16,), jnp.int32)
        trash16 = jnp.full((16,), _R, jnp.int32)
        sd = [sd0, sd1]
        ss = [ss0, ss1]
        dbuf = [dbuf0, dbuf1]
        sbuf = [sbuf0, sbuf1]

        def do_chunk(off):
            cps = [pltpu.async_copy(
                       ftab_hbm.at[mt_src.at[pl.ds(off + q * _GSUB, _GSUB)]],
                       rows.at[pl.ds(q * _GSUB, _GSUB)], gsem)
                   for q in range(_G // _GSUB)]
            for cp in cps:
                cp.wait()

            def group_body(g, _):
                dl16 = mt_dst[pl.ds(off + g * 16, 16)]
                for lane in range(16):
                    rv = rows[g * 16 + lane, :]
                    fx = rv[0]
                    fy = rv[1]
                    fz = rv[2]
                    fw = rv[3]
                    base = dl16[lane] * _OUT
                    for p in range(4):
                        acc_a = (ws[p * 2] * fx + ws[8 + p * 2] * fy
                                 + ws[16 + p * 2] * fz + ws[24 + p * 2] * fw)
                        acc_b = (ws[p * 2 + 1] * fx + ws[8 + p * 2 + 1] * fy
                                 + ws[16 + p * 2 + 1] * fz
                                 + ws[24 + p * 2 + 1] * fw)
                        packed = plsc.pack(acc_a, acc_b,
                                           format=plsc.PackFormat.INTERLEAVED)
                        sl = pl.ds(base + p * 32, 32)
                        m_loc[sl] = jnp.maximum(m_loc[sl], packed)
                return 0

            lax.fori_loop(0, _G // 16, group_body, 0)

        if True:
            rng = wid
            lo = rng * _R
            hi = lo + _R

            def init_body(i, _):
                m_loc[pl.ds(i * 32, 32)] = neg
                return 0

            lax.fori_loop(0, _MLOC // 32, init_body, 0)

            pltpu.async_copy(dst_hbm.at[pl.ds(0, _EBLK)], dbuf[0], sd[0])
            pltpu.async_copy(src_hbm.at[pl.ds(0, _EBLK)], sbuf[0], ss[0])

            def do_block(b, slot, cnt, lo=lo, hi=hi):
                pltpu.make_async_copy(dst_hbm.at[pl.ds(0, _EBLK)],
                                      dbuf[slot], sd[slot]).wait()
                pltpu.make_async_copy(src_hbm.at[pl.ds(0, _EBLK)],
                                      sbuf[slot], ss[slot]).wait()

                @pl.when(b + 1 < nblk)
                def _():
                    nb = (b + 1) * _EBLK
                    pltpu.async_copy(dst_hbm.at[pl.ds(nb, _EBLK)],
                                     dbuf[1 - slot], sd[1 - slot])
                    pltpu.async_copy(src_hbm.at[pl.ds(nb, _EBLK)],
                                     sbuf[1 - slot], ss[1 - slot])

                dref = dbuf[slot]
                sref = sbuf[slot]

                def scan_body(i, cnt):
                    d = dref[pl.ds(i * 16, 16)]
                    s = sref[pl.ds(i * 16, 16)]
                    dl = d - lo
                    msk = dl.astype(jnp.uint32) < jnp.uint32(_R)
                    mi = msk.astype(jnp.int32)
                    csum = plsc.cumsum(mi)
                    pos = cnt + csum - mi
                    plsc.store_scatter(mt_dst, [pos], dl, mask=msk)
                    plsc.store_scatter(mt_src, [pos], s, mask=msk)
                    pc = plsc.all_reduce_population_count(msk)
                    return cnt + pc[0]

                cnt = lax.fori_loop(0, _EBLK // 16, scan_body, cnt,
                                    unroll=8)

                nchunks = cnt // _G

                def chunk_body(ci, _):
                    do_chunk(ci * _G)
                    return 0

                lax.fori_loop(0, nchunks, chunk_body, 0)

                @pl.when(nchunks > 0)
                def _():
                    base = nchunks * _G
                    for i in range(_G // 16):
                        mt_src[pl.ds(i * 16, 16)] = mt_src[pl.ds(base + i * 16, 16)]
                        mt_dst[pl.ds(i * 16, 16)] = mt_dst[pl.ds(base + i * 16, 16)]

                return cnt - nchunks * _G

            def block2(b2, cnt):
                cnt = do_block(b2 * 2, 0, cnt)
                cnt = do_block(b2 * 2 + 1, 1, cnt)
                return cnt

            cnt = lax.fori_loop(0, nblk // 2, block2, jnp.int32(0))

            for k in range(_G // 16):
                plsc.store_scatter(mt_dst, [cnt + k * 16 + iota], trash16)
                plsc.store_scatter(mt_src, [cnt + k * 16 + iota], zero16)

            @pl.when(cnt > 0)
            def _():
                do_chunk(0)

            pltpu.sync_copy(m_loc.at[pl.ds(0, _R * _OUT)],
                            out_hbm.at[pl.ds(rng * _R * _OUT, _R * _OUT)])

    return sc_fn


def _pass_a_body(f_ref, m_ref, a8_ref, p_ref, wg_ref, h_ref, s_ref, acc):
    i = pl.program_id(0)

    @pl.when(i == 0)
    def _():
        acc[...] = jnp.zeros_like(acc)

    fb = f_ref[...]
    xb = fb[:, 3:4]
    a8 = a8_ref[...]
    p = p_ref[...]
    vb = (xb * p[0:1, :] + p[1:2, :]
          - fb[:, 0:1] * a8[0:1, :]
          - fb[:, 1:2] * a8[1:2, :]
          - fb[:, 2:3] * a8[2:3, :])
    mb = m_ref[...].astype(jnp.float32)
    agg = jnp.where(jnp.isfinite(mb), mb + vb, 0.0)
    out = jnp.dot(agg, wg_ref[...], preferred_element_type=jnp.float32) + p[2:3, :]
    conv = xb + out
    h = jnp.where(conv > 0.0, conv, jnp.exp(conv) - 1.0)
    h_ref[...] = h
    rid = i * _BLK + lax.broadcasted_iota(jnp.int32, (_BLK, 1), 0)
    hm = jnp.where(rid < _N, h, 0.0)
    acc[0:1, :] = acc[0:1, :] + jnp.sum(hm, axis=0, keepdims=True)
    acc[1:2, :] = acc[1:2, :] + jnp.sum(hm * hm, axis=0, keepdims=True)

    @pl.when(i == _NB - 1)
    def _():
        s_ref[...] = acc[...]


def _pass_b_body(h_ref, s_ref, bn_ref, y_ref):
    s = s_ref[...]
    mean = s[0:1, :] / _N
    var = s[1:2, :] / _N - mean * mean
    scale = bn_ref[0:1, :] * lax.rsqrt(var + 1e-5)
    shift = bn_ref[1:2, :] - mean * scale
    y_ref[...] = h_ref[...] * scale + shift


def kernel(x, edge_index, edge_attr, Wh, bh, Wf, bf, Wg, bg, bn_w, bn_b):
    pos = edge_index
    src = edge_attr[0]
    dst = edge_attr[1]
    e = src.shape[0]
    nblk = -(-e // (2 * _EBLK)) * 2
    epad = nblk * _EBLK
    src_p = jnp.concatenate([src, jnp.zeros((epad - e,), jnp.int32)])
    dst_p = jnp.concatenate([dst, jnp.full((epad - e,), 1 << 30, jnp.int32)])
    f_tab = jnp.zeros((_NPAD, 16), jnp.float32)
    f_tab = f_tab.at[:_N, :3].set(pos).at[:_N, 3].set(x[:, 0])
    wft4 = Wf.T.astype(jnp.float32).reshape(4, 4, 32)
    wft = jnp.stack([wft4[:, :, 0::2], wft4[:, :, 1::2]], axis=2).reshape(-1)

    m_flat = _seg_max_sc(nblk)(src_p, dst_p, f_tab, wft)
    m = m_flat.reshape(_NPAD, _OUT)

    a3 = Wf[:, :3].T
    a8 = jnp.zeros((8, _OUT), jnp.float32).at[:3].set(a3)
    avec = Wh[:, 0] @ a3
    bvec = bh @ a3 + bf
    p = (jnp.zeros((8, _OUT), jnp.float32)
         .at[0].set(avec).at[1].set(bvec).at[2].set(bg))

    h, stats = pl.pallas_call(
        _pass_a_body,
        grid=(_NB,),
        in_specs=[
            pl.BlockSpec((_BLK, 16), lambda i: (i, 0)),
            pl.BlockSpec((_BLK, _OUT), lambda i: (i, 0)),
            pl.BlockSpec((8, _OUT), lambda i: (0, 0)),
            pl.BlockSpec((8, _OUT), lambda i: (0, 0)),
            pl.BlockSpec((_OUT, _OUT), lambda i: (0, 0)),
        ],
        out_specs=[
            pl.BlockSpec((_BLK, _OUT), lambda i: (i, 0)),
            pl.BlockSpec((8, _OUT), lambda i: (0, 0)),
        ],
        out_shape=[
            jax.ShapeDtypeStruct((_NPAD, _OUT), jnp.float32),
            jax.ShapeDtypeStruct((8, _OUT), jnp.float32),
        ],
        scratch_shapes=[pltpu.VMEM((8, _OUT), jnp.float32)],
    )(f_tab, m, a8, p, Wg.T.astype(jnp.float32))

    bn = jnp.zeros((8, _OUT), jnp.float32).at[0].set(bn_w).at[1].set(bn_b)

    y = pl.pallas_call(
        _pass_b_body,
        grid=(_NB,),
        in_specs=[
            pl.BlockSpec((_BLK, _OUT), lambda i: (i, 0)),
            pl.BlockSpec((8, _OUT), lambda i: (0, 0)),
            pl.BlockSpec((8, _OUT), lambda i: (0, 0)),
        ],
        out_specs=pl.BlockSpec((_BLK, _OUT), lambda i: (i, 0)),
        out_shape=jax.ShapeDtypeStruct((_NPAD, _OUT), jnp.float32),
    )(h, stats, bn)

    return y[:_N]


# speedup vs baseline: 8.6119x; 1.0004x over previous
import functools

import jax
import jax.numpy as jnp
from jax import lax
from jax.experimental import pallas as pl
from jax.experimental.pallas import tpu as pltpu
from jax.experimental.pallas import tpu_sc as plsc

_N = 50000
_OUT = 128
_R = 1568
_NPAD = 32 * _R
_EBLK = 2048
_G = 512
_GSUB = 128
_PBUF = 2592
_MLOC = (_R + 1) * _OUT
_BLK = 512
_NB = _NPAD // _BLK

_mesh = plsc.VectorSubcoreMesh(core_axis_name="c", subcore_axis_name="s")


def _seg_max_sc(nblk):
    @functools.partial(
        pl.kernel,
        mesh=_mesh,
        compiler_params=pltpu.CompilerParams(
            needs_layout_passes=False, use_tc_tiling_on_sc=False),
        out_type=jax.ShapeDtypeStruct((_NPAD * _OUT,), jnp.bfloat16),
        scratch_types=[
            pltpu.VMEM((_MLOC,), jnp.bfloat16),
            pltpu.VMEM((_EBLK,), jnp.int32),
            pltpu.VMEM((_EBLK,), jnp.int32),
            pltpu.VMEM((_EBLK,), jnp.int32),
            pltpu.VMEM((_EBLK,), jnp.int32),
            pltpu.VMEM((_PBUF,), jnp.int32),
            pltpu.VMEM((_PBUF,), jnp.int32),
            pltpu.VMEM((_G, 16), jnp.float32),

            pltpu.VMEM((512,), jnp.float32),
            pltpu.SemaphoreType.DMA,
            pltpu.SemaphoreType.DMA,
            pltpu.SemaphoreType.DMA,
            pltpu.SemaphoreType.DMA,
            pltpu.SemaphoreType.DMA,
        ],
    )
    def sc_fn(src_hbm, dst_hbm, ftab_hbm, wft_hbm, out_hbm,
              m_loc, dbuf0, dbuf1, sbuf0, sbuf1, mt_src, mt_dst, rows, wf_v,
              sd0, sd1, ss0, ss1, gsem):
        wid = lax.axis_index("s") * 2 + lax.axis_index("c")
        pltpu.sync_copy(wft_hbm, wf_v)
        ws = [wf_v[pl.ds(t * 16, 16)] for t in range(32)]
        iota = lax.iota(jnp.int32, 16)
        neg = jnp.full((32,), -jnp.inf, jnp.bfloat16)
        zero16 = jnp.zeros((16,), jnp.int32)
        trash16 = jnp.full((16,), _R, jnp.int32)
        sd = [sd0, sd1]
        ss = [ss0, ss1]
        dbuf = [dbuf0, dbuf1]
        sbuf = [sbuf0, sbuf1]

        def do_chunk(off):
            cps = [pltpu.async_copy(
                       ftab_hbm.at[mt_src.at[pl.ds(off + q * _GSUB, _GSUB)]],
                       rows.at[pl.ds(q * _GSUB, _GSUB)], gsem)
                   for q in range(_G // _GSUB)]
            for cp in cps:
                cp.wait()

            def group_body(g, _):
                dl16 = mt_dst[pl.ds(off + g * 16, 16)]
                for lane in range(16):
                    rv = rows[g * 16 + lane, :]
                    fx = rv[0]
                    fy = rv[1]
                    fz = rv[2]
                    fw = rv[3]
                    base = dl16[lane] * _OUT
                    for p in range(4):
                        acc_a = (ws[p * 2] * fx + ws[8 + p * 2] * fy
                                 + ws[16 + p * 2] * fz + ws[24 + p * 2] * fw)
                        acc_b = (ws[p * 2 + 1] * fx + ws[8 + p * 2 + 1] * fy
                                 + ws[16 + p * 2 + 1] * fz
                                 + ws[24 + p * 2 + 1] * fw)
                        packed = plsc.pack(acc_a, acc_b,
                                           format=plsc.PackFormat.INTERLEAVED)
                        sl = pl.ds(base + p * 32, 32)
                        m_loc[sl] = jnp.maximum(m_loc[sl], packed)
                return 0

            lax.fori_loop(0, _G // 16, group_body, 0)

        if True:
            rng = wid
            lo = rng * _R

            def init_body(i, _):
                m_loc[pl.ds(i * 32, 32)] = neg
                return 0

            lax.fori_loop(0, _MLOC // 32, init_body, 0)

            pltpu.async_copy(dst_hbm.at[pl.ds(0, _EBLK)], dbuf[0], sd[0])
            pltpu.async_copy(src_hbm.at[pl.ds(0, _EBLK)], sbuf[0], ss[0])

            def do_block(b, slot, cnt, lo=lo):
                pltpu.make_async_copy(dst_hbm.at[pl.ds(0, _EBLK)],
                                      dbuf[slot], sd[slot]).wait()
                pltpu.make_async_copy(src_hbm.at[pl.ds(0, _EBLK)],
                                      sbuf[slot], ss[slot]).wait()

                @pl.when(b + 1 < nblk)
                def _():
                    nb = (b + 1) * _EBLK
                    pltpu.async_copy(dst_hbm.at[pl.ds(nb, _EBLK)],
                                     dbuf[1 - slot], sd[1 - slot])
                    pltpu.async_copy(src_hbm.at[pl.ds(nb, _EBLK)],
                                     sbuf[1 - slot], ss[1 - slot])

                dref = dbuf[slot]
                sref = sbuf[slot]

                def scan_body(i, cnt):
                    d = dref[pl.ds(i * 16, 16)]
                    s = sref[pl.ds(i * 16, 16)]
                    dl = d - lo
                    msk = dl.astype(jnp.uint32) < jnp.uint32(_R)
                    mi = msk.astype(jnp.int32)
                    csum = plsc.cumsum(mi)
                    pos = cnt + csum - mi
                    plsc.store_scatter(mt_dst, [pos], dl, mask=msk)
                    plsc.store_scatter(mt_src, [pos], s, mask=msk)
                    pc = plsc.all_reduce_population_count(msk)
                    return cnt + pc[0]

                cnt = lax.fori_loop(0, _EBLK // 16, scan_body, cnt,
                                    unroll=8)

                nchunks = cnt // _G

                def chunk_body(ci, _):
                    do_chunk(ci * _G)
                    return 0

                lax.fori_loop(0, nchunks, chunk_body, 0)

                @pl.when(nchunks > 0)
                def _():
                    base = nchunks * _G
                    for i in range(_G // 16):
                        mt_src[pl.ds(i * 16, 16)] = mt_src[pl.ds(base + i * 16, 16)]
                        mt_dst[pl.ds(i * 16, 16)] = mt_dst[pl.ds(base + i * 16, 16)]

                return cnt - nchunks * _G

            def block2(b2, cnt):
                cnt = do_block(b2 * 2, 0, cnt)
                cnt = do_block(b2 * 2 + 1, 1, cnt)
                return cnt

            cnt = lax.fori_loop(0, nblk // 2, block2, jnp.int32(0))

            for k in range(_G // 16):
                plsc.store_scatter(mt_dst, [cnt + k * 16 + iota], trash16)
                plsc.store_scatter(mt_src, [cnt + k * 16 + iota], zero16)

            @pl.when(cnt > 0)
            def _():
                do_chunk(0)

            pltpu.sync_copy(m_loc.at[pl.ds(0, _R * _OUT)],
                            out_hbm.at[pl.ds(rng * _R * _OUT, _R * _OUT)])

    return sc_fn


def _pass_a_body(f_ref, m_ref, a8_ref, p_ref, wg_ref, h_ref, s_ref, acc):
    i = pl.program_id(0)

    @pl.when(i == 0)
    def _():
        acc[...] = jnp.zeros_like(acc)

    fb = f_ref[...]
    xb = fb[:, 3:4]
    a8 = a8_ref[...]
    p = p_ref[...]
    vb = (xb * p[0:1, :] + p[1:2, :]
          - fb[:, 0:1] * a8[0:1, :]
          - fb[:, 1:2] * a8[1:2, :]
          - fb[:, 2:3] * a8[2:3, :])
    mb = m_ref[...].astype(jnp.float32)
    agg = jnp.where(jnp.isfinite(mb), mb + vb, 0.0)
    out = jnp.dot(agg, wg_ref[...], preferred_element_type=jnp.float32) + p[2:3, :]
    conv = xb + out
    h = jnp.where(conv > 0.0, conv, jnp.exp(conv) - 1.0)
    h_ref[...] = h
    rid = i * _BLK + lax.broadcasted_iota(jnp.int32, (_BLK, 1), 0)
    hm = jnp.where(rid < _N, h, 0.0)
    acc[0:1, :] = acc[0:1, :] + jnp.sum(hm, axis=0, keepdims=True)
    acc[1:2, :] = acc[1:2, :] + jnp.sum(hm * hm, axis=0, keepdims=True)

    @pl.when(i == _NB - 1)
    def _():
        s_ref[...] = acc[...]


def _pass_b_body(h_ref, s_ref, bn_ref, y_ref):
    s = s_ref[...]
    mean = s[0:1, :] / _N
    var = s[1:2, :] / _N - mean * mean
    scale = bn_ref[0:1, :] * lax.rsqrt(var + 1e-5)
    shift = bn_ref[1:2, :] - mean * scale
    y_ref[...] = h_ref[...] * scale + shift


def kernel(x, edge_index, edge_attr, Wh, bh, Wf, bf, Wg, bg, bn_w, bn_b):
    pos = edge_index
    src = edge_attr[0]
    dst = edge_attr[1]
    e = src.shape[0]
    nblk = -(-e // (2 * _EBLK)) * 2
    epad = nblk * _EBLK
    src_p = jnp.concatenate([src, jnp.zeros((epad - e,), jnp.int32)])
    dst_p = jnp.concatenate([dst, jnp.full((epad - e,), 1 << 30, jnp.int32)])
    f_tab = jnp.zeros((_NPAD, 16), jnp.float32)
    f_tab = f_tab.at[:_N, :3].set(pos).at[:_N, 3].set(x[:, 0])
    wft4 = Wf.T.astype(jnp.float32).reshape(4, 4, 32)
    wft = jnp.stack([wft4[:, :, 0::2], wft4[:, :, 1::2]], axis=2).reshape(-1)

    m_flat = _seg_max_sc(nblk)(src_p, dst_p, f_tab, wft)
    m = m_flat.reshape(_NPAD, _OUT)

    a3 = Wf[:, :3].T
    a8 = jnp.zeros((8, _OUT), jnp.float32).at[:3].set(a3)
    avec = Wh[:, 0] @ a3
    bvec = bh @ a3 + bf
    p = (jnp.zeros((8, _OUT), jnp.float32)
         .at[0].set(avec).at[1].set(bvec).at[2].set(bg))

    h, stats = pl.pallas_call(
        _pass_a_body,
        grid=(_NB,),
        in_specs=[
            pl.BlockSpec((_BLK, 16), lambda i: (i, 0)),
            pl.BlockSpec((_BLK, _OUT), lambda i: (i, 0)),
            pl.BlockSpec((8, _OUT), lambda i: (0, 0)),
            pl.BlockSpec((8, _OUT), lambda i: (0, 0)),
            pl.BlockSpec((_OUT, _OUT), lambda i: (0, 0)),
        ],
        out_specs=[
            pl.BlockSpec((_BLK, _OUT), lambda i: (i, 0)),
            pl.BlockSpec((8, _OUT), lambda i: (0, 0)),
        ],
        out_shape=[
            jax.ShapeDtypeStruct((_NPAD, _OUT), jnp.float32),
            jax.ShapeDtypeStruct((8, _OUT), jnp.float32),
        ],
        scratch_shapes=[pltpu.VMEM((8, _OUT), jnp.float32)],
    )(f_tab, m, a8, p, Wg.T.astype(jnp.float32))

    bn = jnp.zeros((8, _OUT), jnp.float32).at[0].set(bn_w).at[1].set(bn_b)

    y = pl.pallas_call(
        _pass_b_body,
        grid=(_NB,),
        in_specs=[
            pl.BlockSpec((_BLK, _OUT), lambda i: (i, 0)),
            pl.BlockSpec((8, _OUT), lambda i: (0, 0)),
            pl.BlockSpec((8, _OUT), lambda i: (0, 0)),
        ],
        out_specs=pl.BlockSpec((_BLK, _OUT), lambda i: (i, 0)),
        out_shape=jax.ShapeDtypeStruct((_NPAD, _OUT), jnp.float32),
    )(h, stats, bn)

    return y[:_N]
